# Initial kernel scaffold; baseline (speedup 1.0000x reference)
import jax
import jax.numpy as jnp
from jax.experimental import pallas as pl


def kernel(x, edge_index, y, W1, b1, g1, be1, W2, b2, g2, be2):
    raise NotImplementedError("write your pallas kernel here")


# speedup vs baseline: 1.6717x; 1.6717x over previous
import functools

import jax
import jax.numpy as jnp
from jax import lax
from jax.experimental import pallas as pl
from jax.experimental.pallas import tpu as pltpu
from jax.experimental.pallas import tpu_sc as plsc

N, C, K = 10000, 128, 32
G = 4
NK = N * K
EPS = 1e-5

NC, NS = 2, 16
NW = NC * NS
NPAD = 10240
NPW = NPAD // NW
CH = 4
CK = CH * K
NCHUNK = NPW // CH
C16 = C // 16


def _conv1_body(xf_ref, w_ref, b_ref, o_ref):
    v = jnp.dot(xf_ref[...], w_ref[...], preferred_element_type=jnp.float32)
    v = v + b_ref[...]
    rows = lax.broadcasted_iota(jnp.int32, (NPAD, C), 0)
    o_ref[...] = jnp.where(rows < N, v, 0.0)


def _conv1(xf_pad, w1bd, b1r):
    return pl.pallas_call(
        _conv1_body,
        out_shape=jax.ShapeDtypeStruct((NPAD, C), jnp.float32),
    )(xf_pad, w1bd, b1r)


def _sc_body(v_hbm, e_hbm, xmax_hbm, s1_hbm, s2_hbm,
             idx_v, rows_v, xm_v, a1_v, a2_v, sem):
    wid = lax.axis_index("s") * NC + lax.axis_index("c")
    base = wid * NPW

    zero = jnp.zeros((16,), jnp.float32)
    init = (zero,) * (2 * C16)

    def chunk_body(i, carry):
        n0 = base + i * CH
        pltpu.sync_copy(e_hbm.at[pl.ds(n0 * K, CK)], idx_v)
        pltpu.async_copy(v_hbm.at[idx_v], rows_v, sem).wait()
        s = list(carry)
        for j in range(CH):
            r0 = j * K
            acc = [rows_v[r0, pl.ds(c * 16, 16)] for c in range(C16)]
            for c in range(C16):
                s[c] = s[c] + acc[c]
                s[C16 + c] = s[C16 + c] + acc[c] * acc[c]

            def k_body(k, kc):
                t = list(kc)
                for c in range(C16):
                    r = rows_v[r0 + k, pl.ds(c * 16, 16)]
                    t[c] = jnp.maximum(t[c], r)
                    t[C16 + c] = t[C16 + c] + r
                    t[2 * C16 + c] = t[2 * C16 + c] + r * r
                return tuple(t)

            res = lax.fori_loop(1, K, k_body, tuple(acc) + tuple(s))
            for c in range(C16):
                xm_v[j, pl.ds(c * 16, 16)] = res[c]
            s = list(res[C16:])
        pltpu.sync_copy(xm_v, xmax_hbm.at[pl.ds(n0, CH)])
        return tuple(s)

    s_fin = lax.fori_loop(0, NCHUNK, chunk_body, init)
    for c in range(C16):
        a1_v[pl.ds(c * 16, 16)] = s_fin[c]
        a2_v[pl.ds(c * 16, 16)] = s_fin[C16 + c]
    pltpu.sync_copy(a1_v, s1_hbm.at[wid])
    pltpu.sync_copy(a2_v, s2_hbm.at[wid])


def _sc_gather_max(v_pad, e_flat):
    mesh = plsc.VectorSubcoreMesh(core_axis_name="c", subcore_axis_name="s")
    fn = functools.partial(
        pl.kernel, mesh=mesh,
        out_type=(jax.ShapeDtypeStruct((NPAD, C), jnp.float32),
                  jax.ShapeDtypeStruct((NW, C), jnp.float32),
                  jax.ShapeDtypeStruct((NW, C), jnp.float32)),
        scratch_types=[pltpu.VMEM((CK,), jnp.int32),
                       pltpu.VMEM((CK, C), jnp.float32),
                       pltpu.VMEM((CH, C), jnp.float32),
                       pltpu.VMEM((C,), jnp.float32),
                       pltpu.VMEM((C,), jnp.float32),
                       pltpu.SemaphoreType.DMA],
    )(_sc_body)
    return fn(v_pad, e_flat)


def _fin_body(xs_ref, xm_ref, s1_ref, s2_ref, w2a_ref, w2b_ref,
              b2_ref, g1_ref, be1_ref, g2_ref, be2_ref, o_ref):
    S1 = jnp.sum(s1_ref[...], axis=0, keepdims=True)
    S2 = jnp.sum(s2_ref[...], axis=0, keepdims=True)
    mean1 = S1 / NK
    var1 = S2 / NK - mean1 * mean1
    a1 = g1_ref[...] * lax.rsqrt(var1 + EPS)
    d1 = be1_ref[...] - a1 * mean1
    xj = jnp.maximum(a1 * xm_ref[...] + d1, 0.0)
    z2 = (jnp.dot(xs_ref[...], w2a_ref[...], preferred_element_type=jnp.float32)
          + jnp.dot(xj, w2b_ref[...], preferred_element_type=jnp.float32)
          + b2_ref[...])
    mean2 = jnp.mean(z2, axis=0, keepdims=True)
    zc = z2 - mean2
    var2 = jnp.mean(zc * zc, axis=0, keepdims=True)
    o_ref[...] = jnp.maximum(
        g2_ref[...] * zc * lax.rsqrt(var2 + EPS) + be2_ref[...], 0.0)


def _finalize(xs, xmax, s1p, s2p, w2a, w2b, b2r, g1r, be1r, g2r, be2r):
    return pl.pallas_call(
        _fin_body,
        out_shape=jax.ShapeDtypeStruct((N, C), jnp.float32),
    )(xs, xmax, s1p, s2p, w2a, w2b, b2r, g1r, be1r, g2r, be2r)


def kernel(x, edge_index, y, W1, b1, g1, be1, W2, b2, g2, be2):
    xf = y.T.reshape(N, C)
    xf_pad = jnp.concatenate([xf, jnp.zeros((NPAD - N, C), jnp.float32)], axis=0)

    cig = C // G
    w1bd = jnp.zeros((C, C), jnp.float32)
    for g in range(G):
        w1bd = w1bd.at[g * cig:(g + 1) * cig, g * cig:(g + 1) * cig].set(
            W1[g * cig:(g + 1) * cig, :].T)
    cig2 = 2 * C // G
    cog2 = C // G
    w2a = jnp.zeros((C, C), jnp.float32)
    w2b = jnp.zeros((C, C), jnp.float32)
    for g in range(G):
        blk = W2[g * cog2:(g + 1) * cog2, :].T
        in0 = g * cig2
        if in0 < C:
            w2a = w2a.at[in0:in0 + cig2, g * cog2:(g + 1) * cog2].set(blk)
        else:
            w2b = w2b.at[in0 - C:in0 - C + cig2, g * cog2:(g + 1) * cog2].set(blk)

    e = edge_index[0]
    e_flat = jnp.concatenate(
        [e, jnp.full((NPAD - N, K), N, jnp.int32)], axis=0).reshape(-1)

    v_pad = _conv1(xf_pad, w1bd, b1.reshape(1, C))
    xmax_pad, s1p, s2p = _sc_gather_max(v_pad, e_flat)

    xs = x.reshape(C, N).T
    out = _finalize(xs, xmax_pad[:N], s1p, s2p, w2a, w2b,
                    b2.reshape(1, C), g1.reshape(1, C), be1.reshape(1, C),
                    g2.reshape(1, C), be2.reshape(1, C))
    return out.T.reshape(1, C, N, 1)


# speedup vs baseline: 2.1115x; 1.2631x over previous
import functools

import jax
import jax.numpy as jnp
from jax import lax
from jax.experimental import pallas as pl
from jax.experimental.pallas import tpu as pltpu
from jax.experimental.pallas import tpu_sc as plsc

N, C, K = 10000, 128, 32
G = 4
NK = N * K
EPS = 1e-5

NC, NS = 2, 16
NW = NC * NS
NPAD = 10240
NPW = NPAD // NW
CH = 4
CK = CH * K
NCHUNK = NPW // CH
C16 = C // 16


def _conv1_body(xf_ref, w_ref, b_ref, o_ref):
    v = jnp.dot(xf_ref[...], w_ref[...], preferred_element_type=jnp.float32)
    v = v + b_ref[...]
    rows = lax.broadcasted_iota(jnp.int32, (NPAD, C), 0)
    o_ref[...] = jnp.where(rows < N, v, 0.0)


def _conv1(xf_pad, w1bd, b1r):
    return pl.pallas_call(
        _conv1_body,
        out_shape=jax.ShapeDtypeStruct((NPAD, C), jnp.float32),
    )(xf_pad, w1bd, b1r)


NBUF = 4


def _sc_body(v_hbm, e_hbm, xmax_hbm, s1_hbm, s2_hbm,
             idx_all, r0, r1, r2, r3, xm_all, a1_v, a2_v,
             sm0, sm1, sm2, sm3):
    rows = [r0, r1, r2, r3]
    sems = [sm0, sm1, sm2, sm3]
    wid = lax.axis_index("s") * NC + lax.axis_index("c")
    base = wid * NPW

    pltpu.sync_copy(e_hbm.at[wid], idx_all)

    def start(i, b):
        pltpu.async_copy(v_hbm.at[idx_all.at[i]], rows[b], sems[b])

    def wait(b):
        pltpu.make_async_copy(v_hbm.at[idx_all.at[0]], rows[b], sems[b]).wait()

    for b in range(NBUF - 1):
        start(b, b)

    zero = jnp.zeros((16,), jnp.float32)
    init = (zero,) * (2 * C16)

    def outer(it, carry):
        s = list(carry)
        for b in range(NBUF):
            i = it * NBUF + b
            nxt = i + NBUF - 1

            @pl.when(nxt < NCHUNK)
            def _():
                start(nxt, (b + NBUF - 1) % NBUF)

            wait(b)
            rv = rows[b]
            for j in range(CH):
                r0_ = j * K
                acc = [rv[r0_, pl.ds(c * 16, 16)] for c in range(C16)]
                for c in range(C16):
                    s[c] = s[c] + acc[c]
                    s[C16 + c] = s[C16 + c] + acc[c] * acc[c]

                def k_body(k, kc):
                    t = list(kc)
                    for c in range(C16):
                        r = rv[r0_ + k, pl.ds(c * 16, 16)]
                        t[c] = jnp.maximum(t[c], r)
                        t[C16 + c] = t[C16 + c] + r
                        t[2 * C16 + c] = t[2 * C16 + c] + r * r
                    return tuple(t)

                res = lax.fori_loop(1, K, k_body, tuple(acc) + tuple(s))
                for c in range(C16):
                    xm_all[i * CH + j, pl.ds(c * 16, 16)] = res[c]
                s = list(res[C16:])
        return tuple(s)

    s_fin = lax.fori_loop(0, NCHUNK // NBUF, outer, init)
    for c in range(C16):
        a1_v[pl.ds(c * 16, 16)] = s_fin[c]
        a2_v[pl.ds(c * 16, 16)] = s_fin[C16 + c]
    pltpu.sync_copy(xm_all, xmax_hbm.at[pl.ds(base, NPW)])
    pltpu.sync_copy(a1_v, s1_hbm.at[wid])
    pltpu.sync_copy(a2_v, s2_hbm.at[wid])


def _sc_gather_max(v_pad, e_r):
    mesh = plsc.VectorSubcoreMesh(core_axis_name="c", subcore_axis_name="s")
    fn = functools.partial(
        pl.kernel, mesh=mesh,
        out_type=(jax.ShapeDtypeStruct((NPAD, C), jnp.float32),
                  jax.ShapeDtypeStruct((NW, C), jnp.float32),
                  jax.ShapeDtypeStruct((NW, C), jnp.float32)),
        scratch_types=[pltpu.VMEM((NCHUNK, CK), jnp.int32)]
                      + [pltpu.VMEM((CK, C), jnp.float32)] * NBUF
                      + [pltpu.VMEM((NPW, C), jnp.float32),
                         pltpu.VMEM((C,), jnp.float32),
                         pltpu.VMEM((C,), jnp.float32)]
                      + [pltpu.SemaphoreType.DMA] * NBUF,
    )(_sc_body)
    return fn(v_pad, e_r)


def _fin_body(xs_ref, xm_ref, s1_ref, s2_ref, w2a_ref, w2b_ref,
              b2_ref, g1_ref, be1_ref, g2_ref, be2_ref, o_ref):
    S1 = jnp.sum(s1_ref[...], axis=0, keepdims=True)
    S2 = jnp.sum(s2_ref[...], axis=0, keepdims=True)
    mean1 = S1 / NK
    var1 = S2 / NK - mean1 * mean1
    a1 = g1_ref[...] * lax.rsqrt(var1 + EPS)
    d1 = be1_ref[...] - a1 * mean1
    xj = jnp.maximum(a1 * xm_ref[...] + d1, 0.0)
    z2 = (jnp.dot(xs_ref[...], w2a_ref[...], preferred_element_type=jnp.float32)
          + jnp.dot(xj, w2b_ref[...], preferred_element_type=jnp.float32)
          + b2_ref[...])
    mean2 = jnp.mean(z2, axis=0, keepdims=True)
    zc = z2 - mean2
    var2 = jnp.mean(zc * zc, axis=0, keepdims=True)
    o_ref[...] = jnp.maximum(
        g2_ref[...] * zc * lax.rsqrt(var2 + EPS) + be2_ref[...], 0.0)


def _finalize(xs, xmax, s1p, s2p, w2a, w2b, b2r, g1r, be1r, g2r, be2r):
    return pl.pallas_call(
        _fin_body,
        out_shape=jax.ShapeDtypeStruct((N, C), jnp.float32),
    )(xs, xmax, s1p, s2p, w2a, w2b, b2r, g1r, be1r, g2r, be2r)


def kernel(x, edge_index, y, W1, b1, g1, be1, W2, b2, g2, be2):
    xf = y.T.reshape(N, C)
    xf_pad = jnp.concatenate([xf, jnp.zeros((NPAD - N, C), jnp.float32)], axis=0)

    cig = C // G
    w1bd = jnp.zeros((C, C), jnp.float32)
    for g in range(G):
        w1bd = w1bd.at[g * cig:(g + 1) * cig, g * cig:(g + 1) * cig].set(
            W1[g * cig:(g + 1) * cig, :].T)
    cig2 = 2 * C // G
    cog2 = C // G
    w2a = jnp.zeros((C, C), jnp.float32)
    w2b = jnp.zeros((C, C), jnp.float32)
    for g in range(G):
        blk = W2[g * cog2:(g + 1) * cog2, :].T
        in0 = g * cig2
        if in0 < C:
            w2a = w2a.at[in0:in0 + cig2, g * cog2:(g + 1) * cog2].set(blk)
        else:
            w2b = w2b.at[in0 - C:in0 - C + cig2, g * cog2:(g + 1) * cog2].set(blk)

    e = edge_index[0]
    e_r = jnp.concatenate(
        [e, jnp.full((NPAD - N, K), N, jnp.int32)],
        axis=0).reshape(NW, NCHUNK, CK)

    v_pad = _conv1(xf_pad, w1bd, b1.reshape(1, C))
    xmax_pad, s1p, s2p = _sc_gather_max(v_pad, e_r)

    xs = x.reshape(C, N).T
    out = _finalize(xs, xmax_pad[:N], s1p, s2p, w2a, w2b,
                    b2.reshape(1, C), g1.reshape(1, C), be1.reshape(1, C),
                    g2.reshape(1, C), be2.reshape(1, C))
    return out.T.reshape(1, C, N, 1)


# speedup vs baseline: 6.9713x; 3.3016x over previous
import functools

import jax
import jax.numpy as jnp
from jax import lax
from jax.experimental import pallas as pl
from jax.experimental.pallas import tpu as pltpu
from jax.experimental.pallas import tpu_sc as plsc

N, C, K = 10000, 128, 32
G = 4
NK = N * K
EPS = 1e-5

NC, NS = 2, 16
NW = NC * NS
NPAD = 10240
NPW = NPAD // NW
CH = 4
CK = CH * K
NCHUNK = NPW // CH
C16 = C // 16
VROWS = N + 8


def _conv1_body(xf_ref, w_ref, b_ref, o_ref):
    v = jnp.dot(xf_ref[...], w_ref[...], preferred_element_type=jnp.float32)
    v = v + b_ref[...]
    rows = lax.broadcasted_iota(jnp.int32, (VROWS, C), 0)
    o_ref[...] = jnp.where(rows < N, v, 0.0)


def _conv1(xf_pad, w1bd, b1r):
    return pl.pallas_call(
        _conv1_body,
        out_shape=jax.ShapeDtypeStruct((VROWS, C), jnp.float32),
    )(xf_pad, w1bd, b1r)


NBUF = 2
XMB = 4


def _sc_body(v_hbm, e_hbm, xmax_hbm, s1_hbm, s2_hbm,
             idx_all, v_sh, r0, r1, x0, x1, x2, x3, a1_v, a2_v,
             sm0, sm1, xs0, xs1, xs2, xs3):
    rows = [r0, r1]
    sems = [sm0, sm1]
    xms = [x0, x1, x2, x3]
    xsems = [xs0, xs1, xs2, xs3]
    sid = lax.axis_index("s")
    wid = sid * NC + lax.axis_index("c")
    base = wid * NPW

    @pl.when(sid == 0)
    def _():
        pltpu.sync_copy(v_hbm, v_sh)

    pltpu.sync_copy(e_hbm.at[wid], idx_all)
    plsc.subcore_barrier()

    def start(i, b):
        pltpu.async_copy(v_sh.at[idx_all.at[i]], rows[b], sems[b])

    def wait(b):
        pltpu.make_async_copy(v_sh.at[idx_all.at[0]], rows[b], sems[b]).wait()

    def xm_wait(xb):
        pltpu.make_async_copy(
            xms[xb], xmax_hbm.at[pl.ds(base, CH)], xsems[xb]).wait()

    start(0, 0)

    zero = jnp.zeros((16,), jnp.float32)
    init = (zero,) * (2 * C16)

    def outer(it, carry):
        s = list(carry)
        for xb in range(XMB):
            i = it * XMB + xb
            b = xb % NBUF

            @pl.when(i + 1 < NCHUNK)
            def _():
                start(i + 1, (b + 1) % NBUF)

            wait(b)

            @pl.when(it > 0)
            def _():
                xm_wait(xb)

            rv = rows[b]
            for j in range(CH):
                r0_ = j * K
                acc = [rv[r0_, pl.ds(c * 16, 16)] for c in range(C16)]
                for c in range(C16):
                    s[c] = s[c] + acc[c]
                    s[C16 + c] = s[C16 + c] + acc[c] * acc[c]

                def k_body(k, kc):
                    t = list(kc)
                    for c in range(C16):
                        r = rv[r0_ + k, pl.ds(c * 16, 16)]
                        t[c] = jnp.maximum(t[c], r)
                        t[C16 + c] = t[C16 + c] + r
                        t[2 * C16 + c] = t[2 * C16 + c] + r * r
                    return tuple(t)

                res = lax.fori_loop(1, K, k_body, tuple(acc) + tuple(s))
                for c in range(C16):
                    xms[xb][j, pl.ds(c * 16, 16)] = res[c]
                s = list(res[C16:])
            pltpu.async_copy(
                xms[xb], xmax_hbm.at[pl.ds(base + i * CH, CH)], xsems[xb])
        return tuple(s)

    s_fin = lax.fori_loop(0, NCHUNK // XMB, outer, init)
    for xb in range(XMB):
        xm_wait(xb)
    for c in range(C16):
        a1_v[pl.ds(c * 16, 16)] = s_fin[c]
        a2_v[pl.ds(c * 16, 16)] = s_fin[C16 + c]
    pltpu.sync_copy(a1_v, s1_hbm.at[wid])
    pltpu.sync_copy(a2_v, s2_hbm.at[wid])


def _sc_gather_max(v_pad, e_r):
    mesh = plsc.VectorSubcoreMesh(core_axis_name="c", subcore_axis_name="s")
    fn = functools.partial(
        pl.kernel, mesh=mesh,
        out_type=(jax.ShapeDtypeStruct((NPAD, C), jnp.float32),
                  jax.ShapeDtypeStruct((NW, C), jnp.float32),
                  jax.ShapeDtypeStruct((NW, C), jnp.float32)),
        scratch_types=[pltpu.VMEM((NCHUNK, CK), jnp.int32),
                       pltpu.VMEM_SHARED((VROWS, C), jnp.float32)]
                      + [pltpu.VMEM((CK, C), jnp.float32)] * NBUF
                      + [pltpu.VMEM((CH, C), jnp.float32)] * XMB
                      + [pltpu.VMEM((C,), jnp.float32),
                         pltpu.VMEM((C,), jnp.float32)]
                      + [pltpu.SemaphoreType.DMA] * (NBUF + XMB),
    )(_sc_body)
    return fn(v_pad, e_r)


def _fin_body(xs_ref, xm_ref, s1_ref, s2_ref, w2a_ref, w2b_ref,
              b2_ref, g1_ref, be1_ref, g2_ref, be2_ref, o_ref):
    S1 = jnp.sum(s1_ref[...], axis=0, keepdims=True)
    S2 = jnp.sum(s2_ref[...], axis=0, keepdims=True)
    mean1 = S1 / NK
    var1 = S2 / NK - mean1 * mean1
    a1 = g1_ref[...] * lax.rsqrt(var1 + EPS)
    d1 = be1_ref[...] - a1 * mean1
    xj = jnp.maximum(a1 * xm_ref[...] + d1, 0.0)
    z2 = (jnp.dot(xs_ref[...], w2a_ref[...], preferred_element_type=jnp.float32)
          + jnp.dot(xj, w2b_ref[...], preferred_element_type=jnp.float32)
          + b2_ref[...])
    mean2 = jnp.mean(z2, axis=0, keepdims=True)
    zc = z2 - mean2
    var2 = jnp.mean(zc * zc, axis=0, keepdims=True)
    o_ref[...] = jnp.maximum(
        g2_ref[...] * zc * lax.rsqrt(var2 + EPS) + be2_ref[...], 0.0)


def _finalize(xs, xmax, s1p, s2p, w2a, w2b, b2r, g1r, be1r, g2r, be2r):
    return pl.pallas_call(
        _fin_body,
        out_shape=jax.ShapeDtypeStruct((N, C), jnp.float32),
    )(xs, xmax, s1p, s2p, w2a, w2b, b2r, g1r, be1r, g2r, be2r)


def kernel(x, edge_index, y, W1, b1, g1, be1, W2, b2, g2, be2):
    xf = y.T.reshape(N, C)
    xf_pad = jnp.concatenate([xf, jnp.zeros((VROWS - N, C), jnp.float32)], axis=0)

    cig = C // G
    w1bd = jnp.zeros((C, C), jnp.float32)
    for g in range(G):
        w1bd = w1bd.at[g * cig:(g + 1) * cig, g * cig:(g + 1) * cig].set(
            W1[g * cig:(g + 1) * cig, :].T)
    cig2 = 2 * C // G
    cog2 = C // G
    w2a = jnp.zeros((C, C), jnp.float32)
    w2b = jnp.zeros((C, C), jnp.float32)
    for g in range(G):
        blk = W2[g * cog2:(g + 1) * cog2, :].T
        in0 = g * cig2
        if in0 < C:
            w2a = w2a.at[in0:in0 + cig2, g * cog2:(g + 1) * cog2].set(blk)
        else:
            w2b = w2b.at[in0 - C:in0 - C + cig2, g * cog2:(g + 1) * cog2].set(blk)

    e = edge_index[0]
    e_r = jnp.concatenate(
        [e, jnp.full((NPAD - N, K), N, jnp.int32)],
        axis=0).reshape(NW, NCHUNK, CK)

    v_pad = _conv1(xf_pad, w1bd, b1.reshape(1, C))
    xmax_pad, s1p, s2p = _sc_gather_max(v_pad, e_r)

    xs = x.reshape(C, N).T
    out = _finalize(xs, xmax_pad[:N], s1p, s2p, w2a, w2b,
                    b2.reshape(1, C), g1.reshape(1, C), be1.reshape(1, C),
                    g2.reshape(1, C), be2.reshape(1, C))
    return out.T.reshape(1, C, N, 1)
